# baseline (speedup 1.0000x reference)
import functools

import jax
import jax.numpy as jnp
import numpy as np
from jax import lax
from jax.experimental import pallas as pl
from jax.experimental.pallas import tpu as pltpu
from jax.experimental.pallas import tpu_sc as plsc

N = 10000
E = 320000
F_IN = 128
HID = 8
OUT = 2

NC = 2
NS = 16
C = 128
CH = 84
R = 4
NG = CH // R
EPT = CH * C
EPAD = NC * NS * EPT
CD = 256
CHD = EPT // CD
NP = 10240
RPT = NP // NS
PR = NP // 16
BLK = 1024

_S_np = np.zeros((8, 128, 128), np.float32)
for _a in range(8):
    for _l in range(128):
        _S_np[_a, 16 * _a + _l // 8, _l] = 1.0
_P_np = np.zeros((8, PR, 80), np.float32)
for _a in range(8):
    for _q in range(80):
        _P_np[_a, 8 * _q + _a, _q] = 1.0


def _mesh():
    return plsc.VectorSubcoreMesh(
        core_axis_name="c", subcore_axis_name="s",
        num_cores=NC, num_subcores=NS)


@functools.partial(
    pl.kernel,
    out_type=jax.ShapeDtypeStruct((NC, NP), jnp.float32),
    mesh=_mesh(),
    compiler_params=pltpu.CompilerParams(use_tc_tiling_on_sc=False),
    scratch_types=[
        pltpu.VMEM((CHD, CD), jnp.int32),
        pltpu.VMEM((CD,), jnp.float32),
        pltpu.VMEM((RPT,), jnp.float32),
        pltpu.VMEM_SHARED((NP,), jnp.float32),
    ],
)
def _sc_degree(dstp, onesc, zrow, out, idx_v, ones_v, row_v, acc_sh):
    c = lax.axis_index("c")
    s = lax.axis_index("s")
    pltpu.sync_copy(dstp.at[c, s], idx_v)
    pltpu.sync_copy(onesc, ones_v)
    pltpu.sync_copy(zrow, row_v)
    pltpu.sync_copy(row_v, acc_sh.at[pl.ds(s * RPT, RPT)])
    plsc.subcore_barrier()

    def body(j, carry):
        pltpu.sync_copy(ones_v, acc_sh.at[idx_v.at[j]], add=True)
        return carry

    lax.fori_loop(0, CHD, body, 0)
    plsc.subcore_barrier()
    pltpu.sync_copy(acc_sh.at[pl.ds(s * RPT, RPT)], row_v)
    pltpu.sync_copy(row_v, out.at[c, pl.ds(s * RPT, RPT)])


@functools.partial(
    pl.kernel,
    out_type=jax.ShapeDtypeStruct((NC, NP, HID), jnp.float32),
    mesh=_mesh(),
    compiler_params=pltpu.CompilerParams(use_tc_tiling_on_sc=False),
    scratch_types=[
        pltpu.VMEM((CH, C), jnp.int32),
        pltpu.VMEM((CH, C), jnp.int32),
        pltpu.VMEM((R, C, HID), jnp.float32),
        pltpu.VMEM((RPT, HID), jnp.float32),
        pltpu.VMEM_SHARED((NP, HID), jnp.float32),
    ] + [pltpu.SemaphoreType.DMA] * R,
)
def _sc_agg(srcp, dstp, g, zrow, out,
            src_v, dst_v, rows_v, buf_v, acc_sh, *sems):
    c = lax.axis_index("c")
    s = lax.axis_index("s")
    pltpu.sync_copy(srcp.at[c, s], src_v)
    pltpu.sync_copy(dstp.at[c, s], dst_v)
    for b in range(R):
        pltpu.async_copy(g.at[src_v.at[b]], rows_v.at[b], sems[b])
    pltpu.sync_copy(zrow, buf_v)
    pltpu.sync_copy(buf_v, acc_sh.at[pl.ds(s * RPT, RPT)])
    plsc.subcore_barrier()

    def body(gi, carry):
        for b in range(R):
            j = gi * R + b
            pltpu.make_async_copy(
                g.at[src_v.at[b]], rows_v.at[b], sems[b]).wait()
            pltpu.sync_copy(rows_v.at[b], acc_sh.at[dst_v.at[j]], add=True)
            pltpu.async_copy(g.at[src_v.at[j + R]], rows_v.at[b], sems[b])
        return carry

    lax.fori_loop(0, NG - 1, body, 0)
    for b in range(R):
        j = (NG - 1) * R + b
        pltpu.make_async_copy(
            g.at[src_v.at[b]], rows_v.at[b], sems[b]).wait()
        pltpu.sync_copy(rows_v.at[b], acc_sh.at[dst_v.at[j]], add=True)
    plsc.subcore_barrier()
    pltpu.sync_copy(acc_sh.at[pl.ds(s * RPT, RPT)], buf_v)
    pltpu.sync_copy(buf_v, out.at[c, pl.ds(s * RPT, RPT)])


def _tc1_body(x_ref, w1_ref, degn_ref, degv_ref, p_ref, s_ref,
              g1_ref, dw_ref):
    deg = degn_ref[0, :] + degn_ref[1, :]
    dinv = lax.rsqrt(deg)[:, None]
    h = jnp.dot(x_ref[...], w1_ref[...],
                preferred_element_type=jnp.float32)
    g1_ref[...] = h * dinv
    dv = lax.rsqrt(degv_ref[0] + degv_ref[1])
    acc = jnp.zeros((BLK // 16, 128), jnp.float32)
    for a in range(8):
        y = jnp.dot(dv, s_ref[a], preferred_element_type=jnp.float32)
        acc = acc + jnp.dot(p_ref[a], y, preferred_element_type=jnp.float32)
    dw_ref[...] = acc


def _tc1(x, w1, degn, degv, pmat, smat):
    return pl.pallas_call(
        _tc1_body,
        grid=(NP // BLK,),
        in_specs=[
            pl.BlockSpec((BLK, F_IN), lambda i: (i, 0)),
            pl.BlockSpec((F_IN, HID), lambda i: (0, 0)),
            pl.BlockSpec((NC, BLK), lambda i: (0, i)),
            pl.BlockSpec((NC, 80, 128), lambda i: (0, 0, 0)),
            pl.BlockSpec((8, BLK // 16, 80), lambda i: (0, i, 0)),
            pl.BlockSpec((8, 128, 128), lambda i: (0, 0, 0)),
        ],
        out_specs=[
            pl.BlockSpec((BLK, HID), lambda i: (i, 0)),
            pl.BlockSpec((BLK // 16, 128), lambda i: (i, 0)),
        ],
        out_shape=[
            jax.ShapeDtypeStruct((NP, HID), jnp.float32),
            jax.ShapeDtypeStruct((PR, 128), jnp.float32),
        ],
    )(x, w1, degn, degv, pmat, smat)


def _tc2_body(p1_ref, dw_ref, b1_ref, w2_ref, g2_ref):
    dw = dw_ref[...]
    h = jnp.maximum((p1_ref[0] + p1_ref[1]) * dw + b1_ref[...], 0.0)
    h2 = jnp.dot(h, w2_ref[...], preferred_element_type=jnp.float32)
    g2_ref[...] = h2 * dw


def _tc2(p1, dw, b1t, w2bd):
    return pl.pallas_call(
        _tc2_body,
        in_specs=[
            pl.BlockSpec((NC, PR, 128), lambda: (0, 0, 0)),
            pl.BlockSpec((PR, 128), lambda: (0, 0)),
            pl.BlockSpec((1, 128), lambda: (0, 0)),
            pl.BlockSpec((128, 128), lambda: (0, 0)),
        ],
        out_specs=pl.BlockSpec((PR, 128), lambda: (0, 0)),
        out_shape=jax.ShapeDtypeStruct((PR, 128), jnp.float32),
    )(p1, dw, b1t, w2bd)


def _tc3_body(p2_ref, dw_ref, b2_ref, z_ref):
    z_ref[...] = (p2_ref[0] + p2_ref[1]) * dw_ref[...] + b2_ref[...]


def _tc3(p2, dw, b2t):
    return pl.pallas_call(
        _tc3_body,
        in_specs=[
            pl.BlockSpec((NC, PR, 128), lambda: (0, 0, 0)),
            pl.BlockSpec((PR, 128), lambda: (0, 0)),
            pl.BlockSpec((1, 128), lambda: (0, 0)),
        ],
        out_specs=pl.BlockSpec((PR, 128), lambda: (0, 0)),
        out_shape=jax.ShapeDtypeStruct((PR, 128), jnp.float32),
    )(p2, dw, b2t)


def kernel(x, edge_index, W1, b1, W2, b2):
    f32 = jnp.float32
    src = edge_index[0].astype(jnp.int32)
    dst = edge_index[1].astype(jnp.int32)
    loop = jnp.arange(N, dtype=jnp.int32)
    npad = EPAD - E - N
    padi = jnp.full((npad,), N, jnp.int32)
    padd = N + jnp.arange(npad, dtype=jnp.int32) % (NP - N)
    srcp = jnp.concatenate([src, loop, padi]).reshape(NC, NS, CH, C)
    dstp = jnp.concatenate([dst, loop, padd]).reshape(NC, NS, CH, C)
    dstp_deg = dstp.reshape(NC, NS, CHD, CD)

    onesc = jnp.ones((CD,), f32)
    degp = _sc_degree(dstp_deg, onesc, jnp.zeros((RPT,), f32))
    degv = degp.reshape(NC, 80, 128)

    pmat = jnp.asarray(_P_np)
    smat = jnp.asarray(_S_np)
    g1, dw = _tc1(x.astype(f32), W1.astype(f32), degp, degv, pmat, smat)

    zrow8 = jnp.zeros((RPT, HID), f32)
    p1 = _sc_agg(srcp, dstp, g1, zrow8)

    w2pad = jnp.zeros((HID, HID), f32).at[:, :OUT].set(W2.astype(f32))
    eye16 = jnp.eye(16, dtype=f32)
    w2bd = jnp.reshape(
        eye16[:, None, :, None] * w2pad[None, :, None, :], (128, 128))
    b1t = jnp.tile(b1.astype(f32), 16).reshape(1, 128)
    b2pad = jnp.zeros((HID,), f32).at[:OUT].set(b2.astype(f32))
    b2t = jnp.tile(b2pad, 16).reshape(1, 128)

    g2 = _tc2(p1.reshape(NC, PR, 128), dw, b1t, w2bd)
    p2 = _sc_agg(srcp, dstp, g2.reshape(NP, HID), zrow8)
    zpk = _tc3(p2.reshape(NC, PR, 128), dw, b2t)
    return zpk.reshape(NP, HID)[:N, :OUT]


# speedup vs baseline: 1.5060x; 1.5060x over previous
import functools

import jax
import jax.numpy as jnp
import numpy as np
from jax import lax
from jax.experimental import pallas as pl
from jax.experimental.pallas import tpu as pltpu
from jax.experimental.pallas import tpu_sc as plsc

N = 10000
E = 320000
F_IN = 128
HID = 8
OUT = 2

NC = 2
NS = 16
C = 128
CH = 81
R = 3
NG = CH // R
EPT = CH * C
EPAD = NC * NS * EPT
CD = 192
CHD = EPT // CD
NP = 10240
RPT = NP // NS
PR = NP // 16
BLK = 1024

_S_np = np.zeros((8, 128, 128), np.float32)
for _a in range(8):
    for _l in range(128):
        _S_np[_a, 16 * _a + _l // 8, _l] = 1.0
_P_np = np.zeros((8, PR, 80), np.float32)
for _a in range(8):
    for _q in range(80):
        _P_np[_a, 8 * _q + _a, _q] = 1.0


def _mesh():
    return plsc.VectorSubcoreMesh(
        core_axis_name="c", subcore_axis_name="s",
        num_cores=NC, num_subcores=NS)


@functools.partial(
    pl.kernel,
    out_type=jax.ShapeDtypeStruct((NC, NP), jnp.float32),
    mesh=_mesh(),
    compiler_params=pltpu.CompilerParams(use_tc_tiling_on_sc=False),
    scratch_types=[
        pltpu.VMEM((CHD, CD), jnp.int32),
        pltpu.VMEM((CD,), jnp.float32),
        pltpu.VMEM((RPT,), jnp.float32),
        pltpu.VMEM_SHARED((NP,), jnp.float32),
    ],
)
def _sc_degree(dstp, onesc, zrow, out, idx_v, ones_v, row_v, acc_sh):
    c = lax.axis_index("c")
    s = lax.axis_index("s")
    pltpu.sync_copy(dstp.at[c, s], idx_v)
    pltpu.sync_copy(onesc, ones_v)
    pltpu.sync_copy(zrow, row_v)
    pltpu.sync_copy(row_v, acc_sh.at[pl.ds(s * RPT, RPT)])
    plsc.subcore_barrier()

    def body(j, carry):
        pltpu.sync_copy(ones_v, acc_sh.at[idx_v.at[j]], add=True)
        return carry

    lax.fori_loop(0, CHD, body, 0)
    plsc.subcore_barrier()
    pltpu.sync_copy(acc_sh.at[pl.ds(s * RPT, RPT)], row_v)
    pltpu.sync_copy(row_v, out.at[c, pl.ds(s * RPT, RPT)])


@functools.partial(
    pl.kernel,
    out_type=jax.ShapeDtypeStruct((NC, NP, HID), jnp.float32),
    mesh=_mesh(),
    compiler_params=pltpu.CompilerParams(use_tc_tiling_on_sc=False),
    scratch_types=[
        pltpu.VMEM((CH, C), jnp.int32),
        pltpu.VMEM((CH, C), jnp.int32),
        pltpu.VMEM((R, C, HID), jnp.float32),
        pltpu.VMEM((RPT, HID), jnp.float32),
        pltpu.VMEM_SHARED((NP, HID), jnp.float32),
    ] + [pltpu.SemaphoreType.DMA] * R,
)
def _sc_agg(srcp, dstp, g, zrow, out,
            src_v, dst_v, rows_v, buf_v, acc_sh, *sems):
    c = lax.axis_index("c")
    s = lax.axis_index("s")
    pltpu.sync_copy(srcp.at[c, s], src_v)
    pltpu.sync_copy(dstp.at[c, s], dst_v)
    for b in range(R):
        pltpu.async_copy(g.at[src_v.at[b]], rows_v.at[b], sems[b])
    pltpu.sync_copy(zrow, buf_v)
    pltpu.sync_copy(buf_v, acc_sh.at[pl.ds(s * RPT, RPT)])
    plsc.subcore_barrier()

    def body(gi, carry):
        for b in range(R):
            j = gi * R + b
            pltpu.make_async_copy(
                g.at[src_v.at[b]], rows_v.at[b], sems[b]).wait()
            pltpu.sync_copy(rows_v.at[b], acc_sh.at[dst_v.at[j]], add=True)
            pltpu.async_copy(g.at[src_v.at[j + R]], rows_v.at[b], sems[b])
        return carry

    lax.fori_loop(0, NG - 1, body, 0)
    for b in range(R):
        j = (NG - 1) * R + b
        pltpu.make_async_copy(
            g.at[src_v.at[b]], rows_v.at[b], sems[b]).wait()
        pltpu.sync_copy(rows_v.at[b], acc_sh.at[dst_v.at[j]], add=True)
    plsc.subcore_barrier()
    pltpu.sync_copy(acc_sh.at[pl.ds(s * RPT, RPT)], buf_v)
    pltpu.sync_copy(buf_v, out.at[c, pl.ds(s * RPT, RPT)])


def _tc1_body(x_ref, w1_ref, degn_ref, degv_ref, p_ref, s_ref,
              g1_ref, dw_ref):
    deg = degn_ref[0, :] + degn_ref[1, :]
    dinv = lax.rsqrt(deg)[:, None]
    h = jnp.dot(x_ref[...], w1_ref[...],
                preferred_element_type=jnp.float32)
    g1_ref[...] = h * dinv
    dv = lax.rsqrt(degv_ref[0] + degv_ref[1])
    acc = jnp.zeros((BLK // 16, 128), jnp.float32)
    for a in range(8):
        y = jnp.dot(dv, s_ref[a], preferred_element_type=jnp.float32)
        acc = acc + jnp.dot(p_ref[a], y, preferred_element_type=jnp.float32)
    dw_ref[...] = acc


def _tc1(x, w1, degn, degv, pmat, smat):
    return pl.pallas_call(
        _tc1_body,
        grid=(NP // BLK,),
        in_specs=[
            pl.BlockSpec((BLK, F_IN), lambda i: (i, 0)),
            pl.BlockSpec((F_IN, HID), lambda i: (0, 0)),
            pl.BlockSpec((NC, BLK), lambda i: (0, i)),
            pl.BlockSpec((NC, 80, 128), lambda i: (0, 0, 0)),
            pl.BlockSpec((8, BLK // 16, 80), lambda i: (0, i, 0)),
            pl.BlockSpec((8, 128, 128), lambda i: (0, 0, 0)),
        ],
        out_specs=[
            pl.BlockSpec((BLK, HID), lambda i: (i, 0)),
            pl.BlockSpec((BLK // 16, 128), lambda i: (i, 0)),
        ],
        out_shape=[
            jax.ShapeDtypeStruct((NP, HID), jnp.float32),
            jax.ShapeDtypeStruct((PR, 128), jnp.float32),
        ],
    )(x, w1, degn, degv, pmat, smat)


def _tc2_body(p1_ref, dw_ref, b1_ref, w2_ref, g2_ref):
    dw = dw_ref[...]
    h = jnp.maximum((p1_ref[0] + p1_ref[1]) * dw + b1_ref[...], 0.0)
    h2 = jnp.dot(h, w2_ref[...], preferred_element_type=jnp.float32)
    g2_ref[...] = h2 * dw


def _tc2(p1, dw, b1t, w2bd):
    return pl.pallas_call(
        _tc2_body,
        in_specs=[
            pl.BlockSpec((NC, PR, 128), lambda: (0, 0, 0)),
            pl.BlockSpec((PR, 128), lambda: (0, 0)),
            pl.BlockSpec((1, 128), lambda: (0, 0)),
            pl.BlockSpec((128, 128), lambda: (0, 0)),
        ],
        out_specs=pl.BlockSpec((PR, 128), lambda: (0, 0)),
        out_shape=jax.ShapeDtypeStruct((PR, 128), jnp.float32),
    )(p1, dw, b1t, w2bd)


def _tc3_body(p2_ref, dw_ref, b2_ref, z_ref):
    z_ref[...] = (p2_ref[0] + p2_ref[1]) * dw_ref[...] + b2_ref[...]


def _tc3(p2, dw, b2t):
    return pl.pallas_call(
        _tc3_body,
        in_specs=[
            pl.BlockSpec((NC, PR, 128), lambda: (0, 0, 0)),
            pl.BlockSpec((PR, 128), lambda: (0, 0)),
            pl.BlockSpec((1, 128), lambda: (0, 0)),
        ],
        out_specs=pl.BlockSpec((PR, 128), lambda: (0, 0)),
        out_shape=jax.ShapeDtypeStruct((PR, 128), jnp.float32),
    )(p2, dw, b2t)


def kernel(x, edge_index, W1, b1, W2, b2):
    f32 = jnp.float32
    src = edge_index[0].astype(jnp.int32)
    dst = edge_index[1].astype(jnp.int32)
    loop = jnp.arange(N, dtype=jnp.int32)
    npad = EPAD - E - N
    padi = N + jnp.arange(npad, dtype=jnp.int32) % (NP - N)
    padd = padi
    srcp = jnp.concatenate([src, loop, padi]).reshape(NC, NS, CH, C)
    dstp = jnp.concatenate([dst, loop, padd]).reshape(NC, NS, CH, C)
    dstp_deg = dstp.reshape(NC, NS, CHD, CD)

    onesc = jnp.ones((CD,), f32)
    degp = _sc_degree(dstp_deg, onesc, jnp.zeros((RPT,), f32))
    degv = degp.reshape(NC, 80, 128)

    pmat = jnp.asarray(_P_np)
    smat = jnp.asarray(_S_np)
    g1, dw = _tc1(x.astype(f32), W1.astype(f32), degp, degv, pmat, smat)

    zrow8 = jnp.zeros((RPT, HID), f32)
    p1 = _sc_agg(srcp, dstp, g1, zrow8)

    w2pad = jnp.zeros((HID, HID), f32).at[:, :OUT].set(W2.astype(f32))
    eye16 = jnp.eye(16, dtype=f32)
    w2bd = jnp.reshape(
        eye16[:, None, :, None] * w2pad[None, :, None, :], (128, 128))
    b1t = jnp.tile(b1.astype(f32), 16).reshape(1, 128)
    b2pad = jnp.zeros((HID,), f32).at[:OUT].set(b2.astype(f32))
    b2t = jnp.tile(b2pad, 16).reshape(1, 128)

    g2 = _tc2(p1.reshape(NC, PR, 128), dw, b1t, w2bd)
    p2 = _sc_agg(srcp, dstp, g2.reshape(NP, HID), zrow8)
    zpk = _tc3(p2.reshape(NC, PR, 128), dw, b2t)
    return zpk.reshape(NP, HID)[:N, :OUT]


# speedup vs baseline: 1.5518x; 1.0304x over previous
import functools

import jax
import jax.numpy as jnp
import numpy as np
from jax import lax
from jax.experimental import pallas as pl
from jax.experimental.pallas import tpu as pltpu
from jax.experimental.pallas import tpu_sc as plsc

N = 10000
E = 320000
F_IN = 128
HID = 8
OUT = 2

NC = 2
NS = 16
C = 128
CH = 81
R = 3
NG = CH // R
EPT = CH * C
EPAD = NC * NS * EPT
EPT_D = E // (NC * NS)
CD = 200
CHD = EPT_D // CD
NP = 10240
RPT = NP // NS
PR = NP // 16
BLK = 1024

_S_np = np.zeros((8, 128, 128), np.float32)
for _a in range(8):
    for _l in range(128):
        _S_np[_a, 16 * _a + _l // 8, _l] = 1.0
_P_np = np.zeros((8, PR, 80), np.float32)
for _a in range(8):
    for _q in range(80):
        _P_np[_a, 8 * _q + _a, _q] = 1.0


def _mesh():
    return plsc.VectorSubcoreMesh(
        core_axis_name="c", subcore_axis_name="s",
        num_cores=NC, num_subcores=NS)


@functools.partial(
    pl.kernel,
    out_type=jax.ShapeDtypeStruct((NC, NP), jnp.float32),
    mesh=_mesh(),
    compiler_params=pltpu.CompilerParams(use_tc_tiling_on_sc=False),
    scratch_types=[
        pltpu.VMEM((CHD, CD), jnp.int32),
        pltpu.VMEM((CD,), jnp.float32),
        pltpu.VMEM((RPT,), jnp.float32),
        pltpu.VMEM_SHARED((NP,), jnp.float32),
    ],
)
def _sc_degree(dstp, onesc, zrow, out, idx_v, ones_v, row_v, acc_sh):
    c = lax.axis_index("c")
    s = lax.axis_index("s")
    pltpu.sync_copy(dstp.at[c, s], idx_v)
    pltpu.sync_copy(onesc, ones_v)
    pltpu.sync_copy(zrow, row_v)
    pltpu.sync_copy(row_v, acc_sh.at[pl.ds(s * RPT, RPT)])
    plsc.subcore_barrier()

    def body(j, carry):
        pltpu.sync_copy(ones_v, acc_sh.at[idx_v.at[j]], add=True)
        return carry

    lax.fori_loop(0, CHD, body, 0)
    plsc.subcore_barrier()
    pltpu.sync_copy(acc_sh.at[pl.ds(s * RPT, RPT)], row_v)
    pltpu.sync_copy(row_v, out.at[c, pl.ds(s * RPT, RPT)])


@functools.partial(
    pl.kernel,
    out_type=jax.ShapeDtypeStruct((NC, NP, HID), jnp.float32),
    mesh=_mesh(),
    compiler_params=pltpu.CompilerParams(use_tc_tiling_on_sc=False),
    scratch_types=[
        pltpu.VMEM((CH, C), jnp.int32),
        pltpu.VMEM((CH, C), jnp.int32),
        pltpu.VMEM((R, C, HID), jnp.float32),
        pltpu.VMEM((RPT, HID), jnp.float32),
        pltpu.VMEM_SHARED((NP, HID), jnp.float32),
    ] + [pltpu.SemaphoreType.DMA] * R,
)
def _sc_agg(srcp, dstp, g, zrow, out,
            src_v, dst_v, rows_v, buf_v, acc_sh, *sems):
    c = lax.axis_index("c")
    s = lax.axis_index("s")
    pltpu.sync_copy(srcp.at[c, s], src_v)
    pltpu.sync_copy(dstp.at[c, s], dst_v)
    for b in range(R):
        pltpu.async_copy(g.at[src_v.at[b]], rows_v.at[b], sems[b])
    pltpu.sync_copy(zrow, buf_v)
    pltpu.sync_copy(buf_v, acc_sh.at[pl.ds(s * RPT, RPT)])
    plsc.subcore_barrier()

    def body(gi, carry):
        for b in range(R):
            j = gi * R + b
            pltpu.make_async_copy(
                g.at[src_v.at[b]], rows_v.at[b], sems[b]).wait()
            pltpu.sync_copy(rows_v.at[b], acc_sh.at[dst_v.at[j]], add=True)
            pltpu.async_copy(g.at[src_v.at[j + R]], rows_v.at[b], sems[b])
        return carry

    lax.fori_loop(0, NG - 1, body, 0)
    for b in range(R):
        j = (NG - 1) * R + b
        pltpu.make_async_copy(
            g.at[src_v.at[b]], rows_v.at[b], sems[b]).wait()
        pltpu.sync_copy(rows_v.at[b], acc_sh.at[dst_v.at[j]], add=True)
    plsc.subcore_barrier()
    pltpu.sync_copy(acc_sh.at[pl.ds(s * RPT, RPT)], buf_v)
    pltpu.sync_copy(buf_v, out.at[c, pl.ds(s * RPT, RPT)])


def _tc1_body(x_ref, w1_ref, degn_ref, g1_ref):
    deg = degn_ref[0, :] + degn_ref[1, :] + 1.0
    dinv = lax.rsqrt(deg)[:, None]
    h = jnp.dot(x_ref[...], w1_ref[...],
                preferred_element_type=jnp.float32)
    g1_ref[...] = h * dinv


def _tc1(x, w1, degn):
    return pl.pallas_call(
        _tc1_body,
        grid=(NP // BLK,),
        in_specs=[
            pl.BlockSpec((BLK, F_IN), lambda i: (i, 0)),
            pl.BlockSpec((F_IN, HID), lambda i: (0, 0)),
            pl.BlockSpec((NC, BLK), lambda i: (0, i)),
        ],
        out_specs=pl.BlockSpec((BLK, HID), lambda i: (i, 0)),
        out_shape=jax.ShapeDtypeStruct((NP, HID), jnp.float32),
    )(x, w1, degn)


def _tcdw_body(degv_ref, p_ref, s_ref, dw_ref):
    dv = lax.rsqrt(degv_ref[0] + degv_ref[1] + 1.0)
    acc = jnp.zeros((PR, 128), jnp.float32)
    for a in range(8):
        y = jnp.dot(dv, s_ref[a], preferred_element_type=jnp.float32)
        acc = acc + jnp.dot(p_ref[a], y, preferred_element_type=jnp.float32)
    dw_ref[...] = acc


def _tcdw(degv, pmat, smat):
    return pl.pallas_call(
        _tcdw_body,
        in_specs=[
            pl.BlockSpec((NC, 80, 128), lambda: (0, 0, 0)),
            pl.BlockSpec((8, PR, 80), lambda: (0, 0, 0)),
            pl.BlockSpec((8, 128, 128), lambda: (0, 0, 0)),
        ],
        out_specs=pl.BlockSpec((PR, 128), lambda: (0, 0)),
        out_shape=jax.ShapeDtypeStruct((PR, 128), jnp.float32),
    )(degv, pmat, smat)


def _tc2_body(p1_ref, dw_ref, b1_ref, w2_ref, g2_ref):
    dw = dw_ref[...]
    h = jnp.maximum((p1_ref[0] + p1_ref[1]) * dw + b1_ref[...], 0.0)
    h2 = jnp.dot(h, w2_ref[...], preferred_element_type=jnp.float32)
    g2_ref[...] = h2 * dw


def _tc2(p1, dw, b1t, w2bd):
    return pl.pallas_call(
        _tc2_body,
        in_specs=[
            pl.BlockSpec((NC, PR, 128), lambda: (0, 0, 0)),
            pl.BlockSpec((PR, 128), lambda: (0, 0)),
            pl.BlockSpec((1, 128), lambda: (0, 0)),
            pl.BlockSpec((128, 128), lambda: (0, 0)),
        ],
        out_specs=pl.BlockSpec((PR, 128), lambda: (0, 0)),
        out_shape=jax.ShapeDtypeStruct((PR, 128), jnp.float32),
    )(p1, dw, b1t, w2bd)


def _tc3_body(p2_ref, dw_ref, b2_ref, z_ref):
    z_ref[...] = (p2_ref[0] + p2_ref[1]) * dw_ref[...] + b2_ref[...]


def _tc3(p2, dw, b2t):
    return pl.pallas_call(
        _tc3_body,
        in_specs=[
            pl.BlockSpec((NC, PR, 128), lambda: (0, 0, 0)),
            pl.BlockSpec((PR, 128), lambda: (0, 0)),
            pl.BlockSpec((1, 128), lambda: (0, 0)),
        ],
        out_specs=pl.BlockSpec((PR, 128), lambda: (0, 0)),
        out_shape=jax.ShapeDtypeStruct((PR, 128), jnp.float32),
    )(p2, dw, b2t)


def kernel(x, edge_index, W1, b1, W2, b2):
    f32 = jnp.float32
    src = edge_index[0].astype(jnp.int32)
    dst = edge_index[1].astype(jnp.int32)
    loop = jnp.arange(N, dtype=jnp.int32)
    npad = EPAD - E - N
    padi = N + jnp.arange(npad, dtype=jnp.int32) % (NP - N)
    padd = padi
    srcp = jnp.concatenate([src, loop, padi]).reshape(NC, NS, CH, C)
    dstp = jnp.concatenate([dst, loop, padd]).reshape(NC, NS, CH, C)

    onesc = jnp.ones((CD,), f32)
    degp = _sc_degree(dst.reshape(NC, NS, CHD, CD), onesc,
                      jnp.zeros((RPT,), f32))
    degv = degp.reshape(NC, 80, 128)

    g1 = _tc1(x.astype(f32), W1.astype(f32), degp)

    zrow8 = jnp.zeros((RPT, HID), f32)
    p1 = _sc_agg(srcp, dstp, g1, zrow8)
    dw = _tcdw(degv, jnp.asarray(_P_np), jnp.asarray(_S_np))

    w2pad = jnp.zeros((HID, HID), f32).at[:, :OUT].set(W2.astype(f32))
    eye16 = jnp.eye(16, dtype=f32)
    w2bd = jnp.reshape(
        eye16[:, None, :, None] * w2pad[None, :, None, :], (128, 128))
    b1t = jnp.tile(b1.astype(f32), 16).reshape(1, 128)
    b2pad = jnp.zeros((HID,), f32).at[:OUT].set(b2.astype(f32))
    b2t = jnp.tile(b2pad, 16).reshape(1, 128)

    g2 = _tc2(p1.reshape(NC, PR, 128), dw, b1t, w2bd)
    p2 = _sc_agg(srcp, dstp, g2.reshape(NP, HID), zrow8)
    zpk = _tc3(p2.reshape(NC, PR, 128), dw, b2t)
    return zpk.reshape(NP, HID)[:N, :OUT]
